# Initial kernel scaffold; baseline (speedup 1.0000x reference)
import jax
import jax.numpy as jnp
from jax.experimental import pallas as pl


def kernel(x_cat, sigma_emb, edge_index, edge_attr, edge_dist, emb0, emb1, emb2, emb3, W_sigma, b_sigma, W_e1, b_e1, W_e2, b_e2, W_f1, b_f1, W_f2, b_f2):
    raise NotImplementedError("write your pallas kernel here")


# speedup vs baseline: 1.0688x; 1.0688x over previous
import functools

import jax
import jax.numpy as jnp
from jax.experimental import pallas as pl
from jax.experimental.pallas import tpu as pltpu

E_BLK = 4000


def _edge_mlp_body(gsrc_ref, gdst_ref, attr_ref, dist_ref, offs_ref,
                   Wa_ref, Wd_ref, We2_ref, be1_ref, be2_ref,
                   Wf1e_ref, Wf1s_ref, Wf1d_ref, bf1_ref, Wf2_ref, bf2_ref,
                   coeff_ref, out_ref):
    n_src = gsrc_ref[:, 0:24]
    p_src = gsrc_ref[:, 24:48]
    n_dst = gdst_ref[:]
    d = dist_ref[:]
    offs = offs_ref[:]
    coeff = coeff_ref[0, 0]
    dd = d - offs
    dist_emb = jnp.exp(coeff * dd * dd)
    e = attr_ref[:] @ Wa_ref[:] + dist_emb @ Wd_ref[:] + p_src + be1_ref[:]
    e = jnp.maximum(e, 0.0)
    e = e @ We2_ref[:] + be2_ref[:]
    h = e @ Wf1e_ref[:] + n_src @ Wf1s_ref[:] + n_dst @ Wf1d_ref[:] + bf1_ref[:]
    h = jnp.maximum(h, 0.0)
    w = h @ Wf2_ref[:] + bf2_ref[:]
    msg = n_dst * w
    one = jnp.ones((msg.shape[0], 1), jnp.float32)
    zero = jnp.zeros((msg.shape[0], 7), jnp.float32)
    out_ref[:] = jnp.concatenate([msg, one, zero], axis=1)


def _edge_mlp(g_src, g_dst, edge_attr, edge_dist, offs, Wa, Wd, We2, be1, be2,
              Wf1e, Wf1s, Wf1d, bf1, Wf2, bf2, coeff):
    n_edges = g_src.shape[0]
    grid = n_edges // E_BLK
    eb = lambda w: pl.BlockSpec((E_BLK, w), lambda i: (i, 0))
    full = lambda a: pl.BlockSpec(a.shape, lambda i: tuple(0 for _ in a.shape))
    return pl.pallas_call(
        _edge_mlp_body,
        grid=(grid,),
        in_specs=[eb(48), eb(24), eb(4), eb(1), full(offs),
                  full(Wa), full(Wd), full(We2), full(be1), full(be2),
                  full(Wf1e), full(Wf1s), full(Wf1d), full(bf1), full(Wf2),
                  full(bf2),
                  pl.BlockSpec(memory_space=pltpu.SMEM)],
        out_specs=eb(32),
        out_shape=jax.ShapeDtypeStruct((n_edges, 32), jnp.float32),
    )(g_src, g_dst, edge_attr, edge_dist, offs, Wa, Wd, We2, be1, be2,
      Wf1e, Wf1s, Wf1d, bf1, Wf2, bf2, coeff)


def kernel(x_cat, sigma_emb, edge_index, edge_attr, edge_dist,
           emb0, emb1, emb2, emb3, W_sigma, b_sigma, W_e1, b_e1,
           W_e2, b_e2, W_f1, b_f1, W_f2, b_f2):
    n_nodes = sigma_emb.shape[0]
    ns = W_sigma.shape[1]
    dist_dim = 32

    node = (emb0[x_cat[:, 0]] + emb1[x_cat[:, 1]] + emb2[x_cat[:, 2]]
            + emb3[x_cat[:, 3]])
    node = node + sigma_emb @ W_sigma + b_sigma
    p_sig = sigma_emb @ W_e1[4:4 + 32]
    src_table = jnp.concatenate([node, p_sig], axis=1)

    src = edge_index[0]
    dst = edge_index[1]

    g_src = jnp.take(src_table, src, axis=0)
    g_dst = jnp.take(node, dst, axis=0)

    lig_max_radius = 5.0
    offs = jnp.linspace(0.0, lig_max_radius, dist_dim)
    coeff = jnp.full((1, 1), -0.5 / (offs[1] - offs[0]) ** 2, jnp.float32)
    Wa = W_e1[0:4]
    Wd = W_e1[36:68]
    Wf1e = W_f1[0:ns]
    Wf1s = W_f1[ns:2 * ns]
    Wf1d = W_f1[2 * ns:3 * ns]
    msg32 = _edge_mlp(g_src, g_dst, edge_attr,
                      edge_dist.reshape(-1, 1), offs.reshape(1, dist_dim),
                      Wa, Wd, W_e2, b_e1.reshape(1, ns), b_e2.reshape(1, ns),
                      Wf1e, Wf1s, Wf1d, b_f1.reshape(1, 3 * ns), W_f2,
                      b_f2.reshape(1, ns), coeff)

    acc = jax.ops.segment_sum(msg32, src, num_segments=n_nodes)
    return acc[:, :24] / jnp.maximum(acc[:, 24:25], 1.0)


# speedup vs baseline: 1.7837x; 1.6689x over previous
import functools

import jax
import jax.numpy as jnp
from jax import lax
from jax.experimental import pallas as pl
from jax.experimental.pallas import tpu as pltpu
from jax.experimental.pallas import tpu_sc as plsc

E_BLK = 4000

_NW = 32
_CH = 128
_K = 8
_SUP = _CH * _K
_NSUP = 50
_EPW = _SUP * _NSUP
_EPAD = _EPW * _NW


def _sc_gather(src_table, node24, srcp2d, dstp2d):
    mesh = plsc.VectorSubcoreMesh(core_axis_name="c", subcore_axis_name="s")

    def body(t48, t24, srcp, dstp, gsrc, gdst,
             idx_s, idx_d, rows48, rows24, sem_g, sem_o):
        nc = 2
        wid = lax.axis_index("s") * nc + lax.axis_index("c")
        row_base = wid * (_EPW // _CH)

        def loop(j, _):
            roff = row_base + j * _K
            eoff = roff * _CH
            pltpu.sync_copy(srcp.at[pl.ds(roff, _K)], idx_s)
            pltpu.sync_copy(dstp.at[pl.ds(roff, _K)], idx_d)
            gs, go = [], []
            for b in range(_K):
                gs.append(pltpu.async_copy(t48.at[idx_s.at[b]], rows48.at[b],
                                           sem_g))
                gs.append(pltpu.async_copy(t24.at[idx_d.at[b]], rows24.at[b],
                                           sem_g))
            for d in gs:
                d.wait()
            for b in range(_K):
                go.append(pltpu.async_copy(
                    rows48.at[b], gsrc.at[pl.ds(eoff + b * _CH, _CH)], sem_o))
                go.append(pltpu.async_copy(
                    rows24.at[b], gdst.at[pl.ds(eoff + b * _CH, _CH)], sem_o))
            for d in go:
                d.wait()
            return 0

        lax.fori_loop(0, _NSUP, loop, 0)

    return pl.kernel(
        body,
        out_type=[jax.ShapeDtypeStruct((_EPAD, 48), jnp.float32),
                  jax.ShapeDtypeStruct((_EPAD, 24), jnp.float32)],
        mesh=mesh,
        scratch_types=[pltpu.VMEM((_K, _CH), jnp.int32),
                       pltpu.VMEM((_K, _CH), jnp.int32),
                       pltpu.VMEM((_K, _CH, 48), jnp.float32),
                       pltpu.VMEM((_K, _CH, 24), jnp.float32),
                       pltpu.SemaphoreType.DMA,
                       pltpu.SemaphoreType.DMA],
        compiler_params=pltpu.CompilerParams(use_tc_tiling_on_sc=False),
    )(src_table, node24, srcp2d, dstp2d)


def _edge_mlp_body(gsrc_ref, gdst_ref, attr_ref, dist_ref, offs_ref,
                   Wa_ref, Wd_ref, We2_ref, be1_ref, be2_ref,
                   Wf1e_ref, Wf1s_ref, Wf1d_ref, bf1_ref, Wf2_ref, bf2_ref,
                   coeff_ref, out_ref):
    n_src = gsrc_ref[:, 0:24]
    p_src = gsrc_ref[:, 24:48]
    n_dst = gdst_ref[:]
    d = dist_ref[:]
    offs = offs_ref[:]
    coeff = coeff_ref[0, 0]
    dd = d - offs
    dist_emb = jnp.exp(coeff * dd * dd)
    e = attr_ref[:] @ Wa_ref[:] + dist_emb @ Wd_ref[:] + p_src + be1_ref[:]
    e = jnp.maximum(e, 0.0)
    e = e @ We2_ref[:] + be2_ref[:]
    h = e @ Wf1e_ref[:] + n_src @ Wf1s_ref[:] + n_dst @ Wf1d_ref[:] + bf1_ref[:]
    h = jnp.maximum(h, 0.0)
    w = h @ Wf2_ref[:] + bf2_ref[:]
    msg = n_dst * w
    one = jnp.ones((msg.shape[0], 1), jnp.float32)
    zero = jnp.zeros((msg.shape[0], 7), jnp.float32)
    out_ref[:] = jnp.concatenate([msg, one, zero], axis=1)


def _edge_mlp(n_edges, g_src, g_dst, edge_attr, edge_dist, offs, Wa, Wd, We2,
              be1, be2, Wf1e, Wf1s, Wf1d, bf1, Wf2, bf2, coeff):
    grid = n_edges // E_BLK
    eb = lambda w: pl.BlockSpec((E_BLK, w), lambda i: (i, 0))
    full = lambda a: pl.BlockSpec(a.shape, lambda i: tuple(0 for _ in a.shape))
    return pl.pallas_call(
        _edge_mlp_body,
        grid=(grid,),
        in_specs=[eb(48), eb(24), eb(4), eb(1), full(offs),
                  full(Wa), full(Wd), full(We2), full(be1), full(be2),
                  full(Wf1e), full(Wf1s), full(Wf1d), full(bf1), full(Wf2),
                  full(bf2),
                  pl.BlockSpec(memory_space=pltpu.SMEM)],
        out_specs=eb(32),
        out_shape=jax.ShapeDtypeStruct((n_edges, 32), jnp.float32),
    )(g_src, g_dst, edge_attr, edge_dist, offs, Wa, Wd, We2, be1, be2,
      Wf1e, Wf1s, Wf1d, bf1, Wf2, bf2, coeff)


def kernel(x_cat, sigma_emb, edge_index, edge_attr, edge_dist,
           emb0, emb1, emb2, emb3, W_sigma, b_sigma, W_e1, b_e1,
           W_e2, b_e2, W_f1, b_f1, W_f2, b_f2):
    n_nodes = sigma_emb.shape[0]
    ns = W_sigma.shape[1]
    dist_dim = 32

    node = (emb0[x_cat[:, 0]] + emb1[x_cat[:, 1]] + emb2[x_cat[:, 2]]
            + emb3[x_cat[:, 3]])
    node = node + sigma_emb @ W_sigma + b_sigma
    p_sig = sigma_emb @ W_e1[4:4 + 32]
    src_table = jnp.concatenate([node, p_sig], axis=1)

    src = edge_index[0].astype(jnp.int32)
    dst = edge_index[1].astype(jnp.int32)
    n_edges = src.shape[0]

    srcp = jnp.zeros((_EPAD,), jnp.int32).at[:n_edges].set(src)
    dstp = jnp.zeros((_EPAD,), jnp.int32).at[:n_edges].set(dst)
    g_src, g_dst = _sc_gather(src_table, node,
                              srcp.reshape(_EPAD // _CH, _CH),
                              dstp.reshape(_EPAD // _CH, _CH))

    lig_max_radius = 5.0
    offs = jnp.linspace(0.0, lig_max_radius, dist_dim)
    coeff = jnp.full((1, 1), -0.5 / (offs[1] - offs[0]) ** 2, jnp.float32)
    Wa = W_e1[0:4]
    Wd = W_e1[36:68]
    Wf1e = W_f1[0:ns]
    Wf1s = W_f1[ns:2 * ns]
    Wf1d = W_f1[2 * ns:3 * ns]
    msg32 = _edge_mlp(n_edges, g_src, g_dst, edge_attr,
                      edge_dist.reshape(-1, 1), offs.reshape(1, dist_dim),
                      Wa, Wd, W_e2, b_e1.reshape(1, ns), b_e2.reshape(1, ns),
                      Wf1e, Wf1s, Wf1d, b_f1.reshape(1, 3 * ns), W_f2,
                      b_f2.reshape(1, ns), coeff)

    acc = jax.ops.segment_sum(msg32, src, num_segments=n_nodes)
    return acc[:, :24] / jnp.maximum(acc[:, 24:25], 1.0)


# speedup vs baseline: 2.4678x; 1.3835x over previous
import functools

import jax
import jax.numpy as jnp
from jax import lax
from jax.experimental import pallas as pl
from jax.experimental.pallas import tpu as pltpu
from jax.experimental.pallas import tpu_sc as plsc

E_BLK = 3200

_NW = 32
_CH = 128
_K = 8
_SUP = _CH * _K
_NSUP = 50
_EPW = _SUP * _NSUP
_EPAD = _EPW * _NW

_NHALF = 50000
_NTR = 50048
_TPS = 16
_EPT = _EPAD // _TPS
_SK = 4
_SSUP = _EPT // (_SK * _CH)
_NVAL = 1600000 // _CH
_STRIPE = _NTR // _TPS


def _sc_gather(src_table, node24, srcp2d, dstp2d):
    mesh = plsc.VectorSubcoreMesh(core_axis_name="c", subcore_axis_name="s")

    def body(t48, t24, srcp, dstp, gsrc, gdst,
             idx_s, idx_d, rows48, rows24, sem_g, sem_o):
        nc = 2
        wid = lax.axis_index("s") * nc + lax.axis_index("c")
        row_base = wid * (_EPW // _CH)

        def loop(j, _):
            roff = row_base + j * _K
            eoff = roff * _CH
            pltpu.sync_copy(srcp.at[pl.ds(roff, _K)], idx_s)
            pltpu.sync_copy(dstp.at[pl.ds(roff, _K)], idx_d)
            gs, go = [], []
            for b in range(_K):
                gs.append(pltpu.async_copy(t48.at[idx_s.at[b]], rows48.at[b],
                                           sem_g))
                gs.append(pltpu.async_copy(t24.at[idx_d.at[b]], rows24.at[b],
                                           sem_g))
            for d in gs:
                d.wait()
            for b in range(_K):
                go.append(pltpu.async_copy(
                    rows48.at[b], gsrc.at[pl.ds(eoff + b * _CH, _CH)], sem_o))
                go.append(pltpu.async_copy(
                    rows24.at[b], gdst.at[pl.ds(eoff + b * _CH, _CH)], sem_o))
            for d in go:
                d.wait()
            return 0

        lax.fori_loop(0, _NSUP, loop, 0)

    return pl.kernel(
        body,
        out_type=[jax.ShapeDtypeStruct((_EPAD, 48), jnp.float32),
                  jax.ShapeDtypeStruct((_EPAD, 24), jnp.float32)],
        mesh=mesh,
        scratch_types=[pltpu.VMEM((_K, _CH), jnp.int32),
                       pltpu.VMEM((_K, _CH), jnp.int32),
                       pltpu.VMEM((_K, _CH, 48), jnp.float32),
                       pltpu.VMEM((_K, _CH, 24), jnp.float32),
                       pltpu.SemaphoreType.DMA,
                       pltpu.SemaphoreType.DMA],
        compiler_params=pltpu.CompilerParams(use_tc_tiling_on_sc=False),
    )(src_table, node24, srcp2d, dstp2d)


def _sc_scatter(msg32, srcp2d):
    mesh = plsc.VectorSubcoreMesh(core_axis_name="c", subcore_axis_name="s")

    def body(msg_hbm, srcp, out_hbm, acc_sh, idx_t, msg_t, zbuf, sem_g):
        c = lax.axis_index("c")
        s = lax.axis_index("s")
        base = c * _NHALF

        def zrow(r, _):
            zbuf[r, pl.ds(0, 16)] = jnp.zeros((16,), jnp.float32)
            zbuf[r, pl.ds(16, 16)] = jnp.zeros((16,), jnp.float32)
            return 0
        lax.fori_loop(0, 128, zrow, 0)
        stripe0 = s * _STRIPE
        def zchunk(k, _):
            pltpu.sync_copy(zbuf, acc_sh.at[pl.ds(stripe0 + k * 128, 128)])
            return 0
        lax.fori_loop(0, 24, zchunk, 0)
        pltpu.sync_copy(zbuf.at[pl.ds(0, 56)],
                        acc_sh.at[pl.ds(stripe0 + 24 * 128, 56)])
        plsc.subcore_barrier()

        iota16 = lax.iota(jnp.int32, 16)

        def sup(j, _):
            row0 = s * (_EPT // _CH) + j * _SK
            pltpu.sync_copy(srcp.at[pl.ds(row0, _SK)], idx_t)
            pltpu.async_copy(
                msg_hbm.at[pl.ds(row0 * _CH, _SK * _CH)], msg_t, sem_g).wait()
            for b in range(_SK):
                cinv = jnp.where(row0 + b < _NVAL, 0, -1)
                for l in range(8):
                    v = idx_t[b, pl.ds(l * 16, 16)]
                    d = v - base
                    oob = ((d | (_NHALF - 1 - d)) >> 31) | cinv
                    trash = _NHALF + ((iota16 + l * 16) & 31)
                    idx_t[b, pl.ds(l * 16, 16)] = (d & ~oob) | (trash & oob)
            for b in range(_SK):
                pltpu.sync_copy(msg_t.at[pl.ds(b * _CH, _CH)],
                                acc_sh.at[idx_t.at[b]], add=True)
            return 0

        lax.fori_loop(0, _SSUP, sup, 0)
        plsc.subcore_barrier()
        pltpu.sync_copy(
            acc_sh.at[pl.ds(stripe0, _STRIPE)],
            out_hbm.at[pl.ds(c * _NTR + stripe0, _STRIPE)])

    return pl.kernel(
        body,
        out_type=jax.ShapeDtypeStruct((2 * _NTR, 32), jnp.float32),
        mesh=mesh,
        scratch_types=[pltpu.VMEM_SHARED((_NTR, 32), jnp.float32),
                       pltpu.VMEM((_SK, _CH), jnp.int32),
                       pltpu.VMEM((_SK * _CH, 32), jnp.float32),
                       pltpu.VMEM((128, 32), jnp.float32),
                       pltpu.SemaphoreType.DMA],
        compiler_params=pltpu.CompilerParams(use_tc_tiling_on_sc=False),
    )(msg32, srcp2d)


def _edge_mlp_body(gsrc_ref, gdst_ref, attr_ref, dist_ref, offs_ref,
                   Wa_ref, Wd_ref, We2_ref, be1_ref, be2_ref,
                   Wf1e_ref, Wf1s_ref, Wf1d_ref, bf1_ref, Wf2_ref, bf2_ref,
                   coeff_ref, out_ref):
    n_src = gsrc_ref[:, 0:24]
    p_src = gsrc_ref[:, 24:48]
    n_dst = gdst_ref[:]
    d = dist_ref[:]
    offs = offs_ref[:]
    coeff = coeff_ref[0, 0]
    dd = d - offs
    dist_emb = jnp.exp(coeff * dd * dd)
    e = attr_ref[:] @ Wa_ref[:] + dist_emb @ Wd_ref[:] + p_src + be1_ref[:]
    e = jnp.maximum(e, 0.0)
    e = e @ We2_ref[:] + be2_ref[:]
    h = e @ Wf1e_ref[:] + n_src @ Wf1s_ref[:] + n_dst @ Wf1d_ref[:] + bf1_ref[:]
    h = jnp.maximum(h, 0.0)
    w = h @ Wf2_ref[:] + bf2_ref[:]
    msg = n_dst * w
    one = jnp.ones((msg.shape[0], 1), jnp.float32)
    zero = jnp.zeros((msg.shape[0], 7), jnp.float32)
    out_ref[:] = jnp.concatenate([msg, one, zero], axis=1)


def _edge_mlp(n_edges, g_src, g_dst, edge_attr, edge_dist, offs, Wa, Wd, We2,
              be1, be2, Wf1e, Wf1s, Wf1d, bf1, Wf2, bf2, coeff):
    grid = n_edges // E_BLK
    eb = lambda w: pl.BlockSpec((E_BLK, w), lambda i: (i, 0))
    full = lambda a: pl.BlockSpec(a.shape, lambda i: tuple(0 for _ in a.shape))
    return pl.pallas_call(
        _edge_mlp_body,
        grid=(grid,),
        in_specs=[eb(48), eb(24), eb(4), eb(1), full(offs),
                  full(Wa), full(Wd), full(We2), full(be1), full(be2),
                  full(Wf1e), full(Wf1s), full(Wf1d), full(bf1), full(Wf2),
                  full(bf2),
                  pl.BlockSpec(memory_space=pltpu.SMEM)],
        out_specs=eb(32),
        out_shape=jax.ShapeDtypeStruct((n_edges, 32), jnp.float32),
    )(g_src, g_dst, edge_attr, edge_dist, offs, Wa, Wd, We2, be1, be2,
      Wf1e, Wf1s, Wf1d, bf1, Wf2, bf2, coeff)


def kernel(x_cat, sigma_emb, edge_index, edge_attr, edge_dist,
           emb0, emb1, emb2, emb3, W_sigma, b_sigma, W_e1, b_e1,
           W_e2, b_e2, W_f1, b_f1, W_f2, b_f2):
    n_nodes = sigma_emb.shape[0]
    ns = W_sigma.shape[1]
    dist_dim = 32

    node = (emb0[x_cat[:, 0]] + emb1[x_cat[:, 1]] + emb2[x_cat[:, 2]]
            + emb3[x_cat[:, 3]])
    node = node + sigma_emb @ W_sigma + b_sigma
    p_sig = sigma_emb @ W_e1[4:4 + 32]
    src_table = jnp.concatenate([node, p_sig], axis=1)

    src = edge_index[0].astype(jnp.int32)
    dst = edge_index[1].astype(jnp.int32)
    n_edges = src.shape[0]

    pad_idx = (jnp.arange(_EPAD - n_edges, dtype=jnp.int32) % n_nodes)
    srcp = jnp.concatenate([src, pad_idx])
    dstp = jnp.concatenate([dst, pad_idx])
    srcp2d = srcp.reshape(_EPAD // _CH, _CH)
    g_src, g_dst = _sc_gather(src_table, node, srcp2d,
                              dstp.reshape(_EPAD // _CH, _CH))

    lig_max_radius = 5.0
    offs = jnp.linspace(0.0, lig_max_radius, dist_dim)
    coeff = jnp.full((1, 1), -0.5 / (offs[1] - offs[0]) ** 2, jnp.float32)
    Wa = W_e1[0:4]
    Wd = W_e1[36:68]
    Wf1e = W_f1[0:ns]
    Wf1s = W_f1[ns:2 * ns]
    Wf1d = W_f1[2 * ns:3 * ns]
    attr_p = jnp.zeros((_EPAD, 4), jnp.float32).at[:n_edges].set(edge_attr)
    dist_p = jnp.zeros((_EPAD, 1), jnp.float32).at[:n_edges, 0].set(edge_dist)
    msg32 = _edge_mlp(_EPAD, g_src, g_dst, attr_p,
                      dist_p, offs.reshape(1, dist_dim),
                      Wa, Wd, W_e2, b_e1.reshape(1, ns), b_e2.reshape(1, ns),
                      Wf1e, Wf1s, Wf1d, b_f1.reshape(1, 3 * ns), W_f2,
                      b_f2.reshape(1, ns), coeff)

    acc2 = _sc_scatter(msg32, srcp2d)
    acc = jnp.concatenate([acc2[:_NHALF], acc2[_NTR:_NTR + _NHALF]], axis=0)
    return acc[:, :24] / jnp.maximum(acc[:, 24:25], 1.0)


# speedup vs baseline: 2.5519x; 1.0341x over previous
import functools

import jax
import jax.numpy as jnp
from jax import lax
from jax.experimental import pallas as pl
from jax.experimental.pallas import tpu as pltpu
from jax.experimental.pallas import tpu_sc as plsc

E_BLK = 3200

_NW = 32
_CH = 128
_K = 8
_SUP = _CH * _K
_NSUP = 50
_EPW = _SUP * _NSUP
_EPAD = _EPW * _NW

_NHALF = 50000
_NTR = 50048
_TPS = 16
_EPT = _EPAD // _TPS
_SK = 4
_SSUP = _EPT // (_SK * _CH)
_NVAL = 1600000 // _CH
_STRIPE = _NTR // _TPS


_GK = 2


def _sc_gather(table128, srcp2d, dstp2d):
    mesh = plsc.VectorSubcoreMesh(core_axis_name="c", subcore_axis_name="s")

    def body(tab, srcp, dstp, comb_s, comb_d, idx_s, idx_d, bufs, bufd,
             sem_g, sem_o):
        nc = 2
        wid = lax.axis_index("s") * nc + lax.axis_index("c")
        row_base = wid * (_EPW // _CH)

        def loop(j, _):
            roff = row_base + j * _GK
            eoff = roff * _CH
            pltpu.sync_copy(srcp.at[pl.ds(roff, _GK)], idx_s)
            pltpu.sync_copy(dstp.at[pl.ds(roff, _GK)], idx_d)
            gs, go = [], []
            for b in range(_GK):
                gs.append(pltpu.async_copy(tab.at[idx_s.at[b]], bufs.at[b],
                                           sem_g))
                gs.append(pltpu.async_copy(tab.at[idx_d.at[b]], bufd.at[b],
                                           sem_g))
            for d in gs:
                d.wait()
            for b in range(_GK):
                go.append(pltpu.async_copy(
                    bufs.at[b], comb_s.at[pl.ds(eoff + b * _CH, _CH)], sem_o))
                go.append(pltpu.async_copy(
                    bufd.at[b], comb_d.at[pl.ds(eoff + b * _CH, _CH)], sem_o))
            for d in go:
                d.wait()
            return 0

        lax.fori_loop(0, _EPW // (_GK * _CH), loop, 0)

    return pl.kernel(
        body,
        out_type=[jax.ShapeDtypeStruct((_EPAD, 128), jnp.float32),
                  jax.ShapeDtypeStruct((_EPAD, 128), jnp.float32)],
        mesh=mesh,
        scratch_types=[pltpu.VMEM((_GK, _CH), jnp.int32),
                       pltpu.VMEM((_GK, _CH), jnp.int32),
                       pltpu.VMEM((_GK, _CH, 128), jnp.float32),
                       pltpu.VMEM((_GK, _CH, 128), jnp.float32),
                       pltpu.SemaphoreType.DMA,
                       pltpu.SemaphoreType.DMA],
        compiler_params=pltpu.CompilerParams(use_tc_tiling_on_sc=True),
    )(table128, srcp2d, dstp2d)


def _sc_scatter(msg32, srcp2d):
    mesh = plsc.VectorSubcoreMesh(core_axis_name="c", subcore_axis_name="s")

    def body(msg_hbm, srcp, out_hbm, acc_sh, idx_t, msg_t, zbuf, sem_g):
        c = lax.axis_index("c")
        s = lax.axis_index("s")
        base = c * _NHALF

        def zrow(r, _):
            zbuf[r, pl.ds(0, 16)] = jnp.zeros((16,), jnp.float32)
            zbuf[r, pl.ds(16, 16)] = jnp.zeros((16,), jnp.float32)
            return 0
        lax.fori_loop(0, 128, zrow, 0)
        stripe0 = s * _STRIPE
        def zchunk(k, _):
            pltpu.sync_copy(zbuf, acc_sh.at[pl.ds(stripe0 + k * 128, 128)])
            return 0
        lax.fori_loop(0, 24, zchunk, 0)
        pltpu.sync_copy(zbuf.at[pl.ds(0, 56)],
                        acc_sh.at[pl.ds(stripe0 + 24 * 128, 56)])
        plsc.subcore_barrier()

        iota16 = lax.iota(jnp.int32, 16)

        def sup(j, _):
            row0 = s * (_EPT // _CH) + j * _SK
            pltpu.sync_copy(srcp.at[pl.ds(row0, _SK)], idx_t)
            pltpu.async_copy(
                msg_hbm.at[pl.ds(row0 * _CH, _SK * _CH)], msg_t, sem_g).wait()
            for b in range(_SK):
                cinv = jnp.where(row0 + b < _NVAL, 0, -1)
                for l in range(8):
                    v = idx_t[b, pl.ds(l * 16, 16)]
                    d = v - base
                    oob = ((d | (_NHALF - 1 - d)) >> 31) | cinv
                    trash = _NHALF + ((iota16 + l * 16) & 31)
                    idx_t[b, pl.ds(l * 16, 16)] = (d & ~oob) | (trash & oob)
            for b in range(_SK):
                pltpu.sync_copy(msg_t.at[pl.ds(b * _CH, _CH)],
                                acc_sh.at[idx_t.at[b]], add=True)
            return 0

        lax.fori_loop(0, _SSUP, sup, 0)
        plsc.subcore_barrier()
        pltpu.sync_copy(
            acc_sh.at[pl.ds(stripe0, _STRIPE)],
            out_hbm.at[pl.ds(c * _NTR + stripe0, _STRIPE)])

    return pl.kernel(
        body,
        out_type=jax.ShapeDtypeStruct((2 * _NTR, 32), jnp.float32),
        mesh=mesh,
        scratch_types=[pltpu.VMEM_SHARED((_NTR, 32), jnp.float32),
                       pltpu.VMEM((_SK, _CH), jnp.int32),
                       pltpu.VMEM((_SK * _CH, 32), jnp.float32),
                       pltpu.VMEM((128, 32), jnp.float32),
                       pltpu.SemaphoreType.DMA],
        compiler_params=pltpu.CompilerParams(use_tc_tiling_on_sc=False),
    )(msg32, srcp2d)


def _edge_mlp_body(combs_ref, combd_ref, attr_ref, dist_ref, offs_ref,
                   Wa_ref, Wd_ref, We2_ref, be1_ref, be2_ref,
                   Wf1e_ref, Wf1s_ref, Wf1d_ref, bf1_ref, Wf2_ref, bf2_ref,
                   coeff_ref, out_ref):
    n_src = combs_ref[:, 0:24]
    p_src = combs_ref[:, 24:48]
    n_dst = combd_ref[:, 0:24]
    d = dist_ref[:]
    offs = offs_ref[:]
    coeff = coeff_ref[0, 0]
    dd = d - offs
    dist_emb = jnp.exp(coeff * dd * dd)
    e = attr_ref[:] @ Wa_ref[:] + dist_emb @ Wd_ref[:] + p_src + be1_ref[:]
    e = jnp.maximum(e, 0.0)
    e = e @ We2_ref[:] + be2_ref[:]
    h = e @ Wf1e_ref[:] + n_src @ Wf1s_ref[:] + n_dst @ Wf1d_ref[:] + bf1_ref[:]
    h = jnp.maximum(h, 0.0)
    w = h @ Wf2_ref[:] + bf2_ref[:]
    msg = n_dst * w
    one = jnp.ones((msg.shape[0], 1), jnp.float32)
    zero = jnp.zeros((msg.shape[0], 7), jnp.float32)
    out_ref[:] = jnp.concatenate([msg, one, zero], axis=1)


def _edge_mlp(n_edges, comb_s, comb_d, edge_attr, edge_dist, offs, Wa, Wd, We2,
              be1, be2, Wf1e, Wf1s, Wf1d, bf1, Wf2, bf2, coeff):
    grid = n_edges // E_BLK
    eb = lambda w: pl.BlockSpec((E_BLK, w), lambda i: (i, 0))
    full = lambda a: pl.BlockSpec(a.shape, lambda i: tuple(0 for _ in a.shape))
    return pl.pallas_call(
        _edge_mlp_body,
        grid=(grid,),
        in_specs=[eb(128), eb(128), eb(4), eb(1), full(offs),
                  full(Wa), full(Wd), full(We2), full(be1), full(be2),
                  full(Wf1e), full(Wf1s), full(Wf1d), full(bf1), full(Wf2),
                  full(bf2),
                  pl.BlockSpec(memory_space=pltpu.SMEM)],
        out_specs=eb(32),
        out_shape=jax.ShapeDtypeStruct((n_edges, 32), jnp.float32),
    )(comb_s, comb_d, edge_attr, edge_dist, offs, Wa, Wd, We2, be1, be2,
      Wf1e, Wf1s, Wf1d, bf1, Wf2, bf2, coeff)


def kernel(x_cat, sigma_emb, edge_index, edge_attr, edge_dist,
           emb0, emb1, emb2, emb3, W_sigma, b_sigma, W_e1, b_e1,
           W_e2, b_e2, W_f1, b_f1, W_f2, b_f2):
    n_nodes = sigma_emb.shape[0]
    ns = W_sigma.shape[1]
    dist_dim = 32

    node = (emb0[x_cat[:, 0]] + emb1[x_cat[:, 1]] + emb2[x_cat[:, 2]]
            + emb3[x_cat[:, 3]])
    node = node + sigma_emb @ W_sigma + b_sigma
    p_sig = sigma_emb @ W_e1[4:4 + 32]
    table128 = jnp.concatenate(
        [node, p_sig, node, jnp.zeros((n_nodes, 56), jnp.float32)], axis=1)

    src = edge_index[0].astype(jnp.int32)
    dst = edge_index[1].astype(jnp.int32)
    n_edges = src.shape[0]

    pad_idx = (jnp.arange(_EPAD - n_edges, dtype=jnp.int32) % n_nodes)
    srcp = jnp.concatenate([src, pad_idx])
    dstp = jnp.concatenate([dst, pad_idx])
    srcp2d = srcp.reshape(_EPAD // _CH, _CH)
    comb_s, comb_d = _sc_gather(table128, srcp2d,
                                dstp.reshape(_EPAD // _CH, _CH))

    lig_max_radius = 5.0
    offs = jnp.linspace(0.0, lig_max_radius, dist_dim)
    coeff = jnp.full((1, 1), -0.5 / (offs[1] - offs[0]) ** 2, jnp.float32)
    Wa = W_e1[0:4]
    Wd = W_e1[36:68]
    Wf1e = W_f1[0:ns]
    Wf1s = W_f1[ns:2 * ns]
    Wf1d = W_f1[2 * ns:3 * ns]
    attr_p = jnp.zeros((_EPAD, 4), jnp.float32).at[:n_edges].set(edge_attr)
    dist_p = jnp.zeros((_EPAD, 1), jnp.float32).at[:n_edges, 0].set(edge_dist)
    msg32 = _edge_mlp(_EPAD, comb_s, comb_d, attr_p,
                      dist_p, offs.reshape(1, dist_dim),
                      Wa, Wd, W_e2, b_e1.reshape(1, ns), b_e2.reshape(1, ns),
                      Wf1e, Wf1s, Wf1d, b_f1.reshape(1, 3 * ns), W_f2,
                      b_f2.reshape(1, ns), coeff)

    acc2 = _sc_scatter(msg32, srcp2d)
    acc = jnp.concatenate([acc2[:_NHALF], acc2[_NTR:_NTR + _NHALF]], axis=0)
    return acc[:, :24] / jnp.maximum(acc[:, 24:25], 1.0)
